# baseline (speedup 1.0000x reference)
import functools

import jax
import jax.numpy as jnp
from jax import lax
from jax.experimental import pallas as pl
from jax.experimental.pallas import tpu as pltpu
from jax.experimental.pallas import tpu_sc as plsc

BATCH = 4096
SEQ = 50
D = 32
B = BATCH * SEQ
VOCAB = (1000000, 100000, 1000)

_info = plsc.get_sparse_core_info()
NC, NS = _info.num_cores, _info.num_subcores
NW = NC * NS
NBLK = SEQ * (BATCH // 128)
BLK_PER_W = NBLK // NW
TILE_W = 8 * 128
ROW_W = 3 * 4 * TILE_W
OUT_WORDS = NBLK * ROW_W


def _sc_embed(u2, i2, c2, wtu, wti, wtc):
    mesh = plsc.VectorSubcoreMesh(core_axis_name="c", subcore_axis_name="s")

    @functools.partial(
        pl.kernel,
        mesh=mesh,
        out_type=jax.ShapeDtypeStruct((OUT_WORDS,), jnp.float32),
        compiler_params=pltpu.CompilerParams(use_tc_tiling_on_sc=False),
        scratch_types=[
            pltpu.VMEM((BLK_PER_W, 128), jnp.int32),
            pltpu.VMEM((BLK_PER_W, 128), jnp.int32),
            pltpu.VMEM((BLK_PER_W, 128), jnp.int32),
            pltpu.VMEM((3, 4096), jnp.int32),
            pltpu.VMEM((3, 4096), jnp.int32),
            pltpu.VMEM((3, 4096), jnp.float32),
            pltpu.VMEM((3, 4096), jnp.float32),
            pltpu.SemaphoreType.DMA,
            [pltpu.SemaphoreType.DMA] * 2,
            [pltpu.SemaphoreType.DMA] * 2,
        ],
    )
    def k(u, i, c, wu, wi, wc, out,
          idx_u, idx_i, idx_c, widxA, widxB, rbA, rbB, sem0, gsems, osems):
        wid = lax.axis_index("s") * NC + lax.axis_index("c")
        blk0 = wid * BLK_PER_W

        s1 = pltpu.async_copy(u.at[pl.ds(blk0, BLK_PER_W)], idx_u, sem0)
        s2 = pltpu.async_copy(i.at[pl.ds(blk0, BLK_PER_W)], idx_i, sem0)
        s3 = pltpu.async_copy(c.at[pl.ds(blk0, BLK_PER_W)], idx_c, sem0)
        s1.wait()
        s2.wait()
        s3.wait()

        tables = (wu, wi, wc)
        idxs = (idx_u, idx_i, idx_c)

        def fill_widx(widx, j):
            for t in range(3):
                vecs = [idxs[t][j, pl.ds(16 * c8, 16)] for c8 in range(8)]
                for ti in range(4):
                    for a in range(8):
                        base = (8 * ti + a) * VOCAB[t]
                        for c8 in range(8):
                            widx[t, pl.ds(ti * 1024 + a * 128 + 16 * c8, 16)] = (
                                vecs[c8] + base
                            )

        def gathers(widx, rb, sem):
            return [
                pltpu.async_copy(tables[t].at[widx.at[t]], rb.at[t], sem)
                for t in range(3)
            ]

        def outs(rb, j, sem, make):
            z = blk0 + j
            s = z // 32
            bb = z - s * 32
            mk = pltpu.make_async_copy if make else pltpu.async_copy
            return [
                mk(
                    rb.at[t, pl.ds(ti * 1024, 1024)],
                    out.at[pl.ds(s * 393216 + (4 * t + ti) * 32768
                                 + bb * 1024, 1024)],
                    sem,
                )
                for t in range(3)
                for ti in range(4)
            ]

        def loop_body(m, carry):
            jA = 2 * m
            jB = 2 * m + 1

            @pl.when(m > 0)
            def _():
                for d in outs(rbA, jA, osems[0], make=True):
                    d.wait()

            fill_widx(widxA, jA)
            gA = gathers(widxA, rbA, gsems[0])

            @pl.when(m > 0)
            def _():
                for d in outs(rbB, jB, osems[1], make=True):
                    d.wait()

            fill_widx(widxB, jB)
            gB = gathers(widxB, rbB, gsems[1])

            for d in gA:
                d.wait()
            outs(rbA, jA, osems[0], make=False)
            for d in gB:
                d.wait()
            outs(rbB, jB, osems[1], make=False)
            return carry

        lax.fori_loop(0, BLK_PER_W // 2, loop_body, 0)
        for d in outs(rbA, BLK_PER_W - 2, osems[0], make=True):
            d.wait()
        for d in outs(rbB, BLK_PER_W - 1, osems[1], make=True):
            d.wait()

    return k(u2, i2, c2, wtu, wti, wtc)


def kernel(user_id, item_id, category, W_user_id, W_item_id, W_category):
    u2 = user_id.astype(jnp.int32).T.reshape(NBLK, 128)
    i2 = item_id.astype(jnp.int32).T.reshape(NBLK, 128)
    c2 = category.astype(jnp.int32).T.reshape(NBLK, 128)
    wtu = W_user_id.T.reshape(D * VOCAB[0])
    wti = W_item_id.T.reshape(D * VOCAB[1])
    wtc = W_category.T.reshape(D * VOCAB[2])
    out1 = _sc_embed(u2, i2, c2, wtu, wti, wtc)
    return (out1.reshape(SEQ, 12, 32, 8, 128)
            .transpose(2, 4, 0, 1, 3)
            .reshape(BATCH, SEQ, 3 * D))


# speedup vs baseline: 3.1131x; 3.1131x over previous
import functools

import jax
import jax.numpy as jnp
from jax import lax
from jax.experimental import pallas as pl
from jax.experimental.pallas import tpu as pltpu
from jax.experimental.pallas import tpu_sc as plsc

BATCH = 4096
SEQ = 50
D = 32
B = BATCH * SEQ

_info = plsc.get_sparse_core_info()
NC, NS = _info.num_cores, _info.num_subcores
NW = NC * NS
OUT_WORDS = SEQ * 12 * 32 * 1024


def _sc_embed(u2, i2, c2, wu, wi, wc):
    mesh = plsc.VectorSubcoreMesh(core_axis_name="c", subcore_axis_name="s")

    @functools.partial(
        pl.kernel,
        mesh=mesh,
        out_type=jax.ShapeDtypeStruct((OUT_WORDS,), jnp.float32),
        compiler_params=pltpu.CompilerParams(use_tc_tiling_on_sc=False,
                                             needs_layout_passes=False),
        scratch_types=[
            pltpu.VMEM((128, SEQ), jnp.int32),
            pltpu.VMEM((128, SEQ), jnp.int32),
            pltpu.VMEM((128, SEQ), jnp.int32),
            pltpu.VMEM((SEQ, 128), jnp.int32),
            pltpu.VMEM((SEQ, 128), jnp.int32),
            pltpu.VMEM((SEQ, 128), jnp.int32),
            [pltpu.VMEM((128, D), jnp.float32)] * 6,
            [pltpu.VMEM((12, 1024), jnp.float32)] * 2,
            pltpu.SemaphoreType.DMA,
            [pltpu.SemaphoreType.DMA] * 2,
            [pltpu.SemaphoreType.DMA] * 2,
        ],
    )
    def k(u, i, c, tu, ti, tc, out,
          su, si, sc, xu, xi, xc, rbufs, tbufs, sem0, gsems, osems):
        wid = lax.axis_index("s") * NC + lax.axis_index("c")
        b0 = wid * 128

        d1 = pltpu.async_copy(u.at[pl.ds(b0, 128)], su, sem0)
        d2 = pltpu.async_copy(i.at[pl.ds(b0, 128)], si, sem0)
        d3 = pltpu.async_copy(c.at[pl.ds(b0, 128)], sc, sem0)
        d1.wait()
        d2.wait()
        d3.wait()

        bvecs = [lax.broadcasted_iota(jnp.int32, (16,), 0) + 16 * q
                 for q in range(8)]

        def tr_body(s, carry):
            for src, dst in ((su, xu), (si, xi), (sc, xc)):
                svec = jnp.full((16,), 0, jnp.int32) + s
                for q in range(8):
                    dst[s, pl.ds(16 * q, 16)] = plsc.load_gather(
                        src, [bvecs[q], svec])
            return carry

        lax.fori_loop(0, SEQ, tr_body, 0)

        tables = (tu, ti, tc)
        xs = (xu, xi, xc)

        def gathers(s, buf, sem):
            return [
                pltpu.async_copy(tables[t].at[xs[t].at[s]], rbufs[3 * buf + t],
                                 sem)
                for t in range(3)
            ]

        def outs(s, buf, sem, make):
            mk = pltpu.make_async_copy if make else pltpu.async_copy
            return [
                mk(tbufs[buf].at[dd],
                   out.at[pl.ds(s * 393216 + dd * 32768 + wid * 1024, 1024)],
                   sem)
                for dd in range(12)
            ]

        def transpose_into(buf):
            tb = tbufs[buf]
            for t in range(3):
                rb = rbufs[3 * buf + t]
                for ti_ in range(4):
                    for a in range(8):
                        dcol = jnp.full((16,), 0, jnp.int32) + (8 * ti_ + a)
                        for q in range(8):
                            tb[4 * t + ti_, pl.ds(a * 128 + 16 * q, 16)] = (
                                plsc.load_gather(rb, [bvecs[q], dcol]))

        def loop_body(m, carry):
            sA = 2 * m
            sB = 2 * m + 1
            gB = gathers(sB, 1, gsems[1])
            for t in range(3):
                pltpu.make_async_copy(
                    tables[t].at[xs[t].at[sA]], rbufs[t], gsems[0]).wait()

            @pl.when(m > 0)
            def _():
                for dsc in outs(sA, 0, osems[0], make=True):
                    dsc.wait()

            transpose_into(0)
            outs(sA, 0, osems[0], make=False)

            @pl.when(m < (SEQ // 2) - 1)
            def _():
                gathers(sA + 2, 0, gsems[0])

            for dsc in gB:
                dsc.wait()

            @pl.when(m > 0)
            def _():
                for dsc in outs(sB, 1, osems[1], make=True):
                    dsc.wait()

            transpose_into(1)
            outs(sB, 1, osems[1], make=False)
            return carry

        gathers(0, 0, gsems[0])
        lax.fori_loop(0, SEQ // 2, loop_body, 0)
        for dsc in outs(SEQ - 2, 0, osems[0], make=True):
            dsc.wait()
        for dsc in outs(SEQ - 1, 1, osems[1], make=True):
            dsc.wait()

    return k(u2, i2, c2, wu, wi, wc)


def kernel(user_id, item_id, category, W_user_id, W_item_id, W_category):
    u2 = user_id.astype(jnp.int32)
    i2 = item_id.astype(jnp.int32)
    c2 = category.astype(jnp.int32)
    out1 = _sc_embed(u2, i2, c2, W_user_id, W_item_id, W_category)
    return (out1.reshape(SEQ, 12, 32, 8, 128)
            .transpose(2, 4, 0, 1, 3)
            .reshape(BATCH, SEQ, 3 * D))
